# baseline (speedup 1.0000x reference)
import functools

import jax
import jax.numpy as jnp
from jax import lax
from jax.experimental import pallas as pl
from jax.experimental.pallas import tpu as pltpu
from jax.experimental.pallas import tpu_sc as plsc

_BATCH = 16384
_NF = 26
_NC = 2
_NS = 16
_NW = _NC * _NS
_RPW = _BATCH // _NW
_IPW = _RPW * _NF
_MINOR = 128
_NROW = _IPW // _MINOR
_CHUNKS = _RPW // 16


def _sc_body(x_hbm, table_hbm, out_hbm, idx_v, val_v, out_v, sem):
    wid = lax.axis_index("s") * _NC + lax.axis_index("c")

    pltpu.sync_copy(x_hbm.at[wid], idx_v)
    pltpu.async_copy(table_hbm.at[idx_v], val_v, sem).wait()

    lane = lax.iota(jnp.int32, 16) * _NF
    zero = jnp.zeros((16,), jnp.int32)

    def chunk_body(c, carry):
        base = c * (16 * _NF)
        acc = jnp.zeros((16,), jnp.float32)
        for f in range(_NF):
            acc = acc + plsc.load_gather(val_v, [lane + (base + f), zero])
        out_v[pl.ds(c * 16, 16)] = acc
        return carry

    lax.fori_loop(0, _CHUNKS, chunk_body, 0)
    pltpu.sync_copy(out_v, out_hbm.at[pl.ds(wid * _RPW, _RPW)])


_sc_call = pl.kernel(
    _sc_body,
    out_type=jax.ShapeDtypeStruct((_BATCH,), jnp.float32),
    mesh=plsc.VectorSubcoreMesh(core_axis_name="c", subcore_axis_name="s"),
    scratch_types=[
        pltpu.VMEM((_IPW,), jnp.int32),
        pltpu.VMEM((_IPW, 1), jnp.float32),
        pltpu.VMEM((_RPW,), jnp.float32),
        pltpu.SemaphoreType.DMA,
    ],
    compiler_params=pltpu.CompilerParams(
        needs_layout_passes=False, use_tc_tiling_on_sc=False
    ),
)


@jax.jit
def kernel(x, table, bias):
    xr = x.reshape(_NW, _IPW)
    out = _sc_call(xr, table)
    return out.reshape(-1, 1) + bias


# speedup vs baseline: 23.4278x; 23.4278x over previous
import functools

import jax
import jax.numpy as jnp
from jax import lax
from jax.experimental import pallas as pl
from jax.experimental.pallas import tpu as pltpu
from jax.experimental.pallas import tpu_sc as plsc

_BATCH = 16384
_NF = 26
_NC = 2
_NS = 16
_NW = _NC * _NS
_RPW = _BATCH // _NW
_IPW = _RPW * _NF
_CHUNKS = _RPW // 16


def _sc_body(xt_hbm, table_hbm, out_hbm, idx_v, val_v, out_v, sem, gsem):
    wid = lax.axis_index("s") * _NC + lax.axis_index("c")
    base = wid * _RPW

    for f in range(_NF):
        pltpu.async_copy(
            xt_hbm.at[f, pl.ds(base, _RPW)],
            idx_v.at[pl.ds(f * _RPW, _RPW)],
            sem,
        )
    for f in range(_NF):
        pltpu.make_async_copy(
            xt_hbm.at[f, pl.ds(base, _RPW)],
            idx_v.at[pl.ds(f * _RPW, _RPW)],
            sem,
        ).wait()

    pltpu.async_copy(table_hbm.at[0].at[idx_v], val_v, gsem).wait()

    def chunk_body(c, carry):
        acc = val_v[pl.ds(c * 16, 16)]
        for f in range(1, _NF):
            acc = acc + val_v[pl.ds(f * _RPW + c * 16, 16)]
        out_v[pl.ds(c * 16, 16)] = acc
        return carry

    lax.fori_loop(0, _CHUNKS, chunk_body, 0)
    pltpu.sync_copy(out_v, out_hbm.at[pl.ds(base, _RPW)])


_sc_call = pl.kernel(
    _sc_body,
    out_type=jax.ShapeDtypeStruct((_BATCH,), jnp.float32),
    mesh=plsc.VectorSubcoreMesh(core_axis_name="c", subcore_axis_name="s"),
    scratch_types=[
        pltpu.VMEM((_IPW,), jnp.int32),
        pltpu.VMEM((_IPW,), jnp.float32),
        pltpu.VMEM((_RPW,), jnp.float32),
        pltpu.SemaphoreType.DMA,
        pltpu.SemaphoreType.DMA,
    ],
    compiler_params=pltpu.CompilerParams(needs_layout_passes=False),
)


@jax.jit
def kernel(x, table, bias):
    out = _sc_call(x.T, table.T)
    return out.reshape(-1, 1) + bias


# speedup vs baseline: 24.4252x; 1.0426x over previous
import functools

import jax
import jax.numpy as jnp
from jax import lax
from jax.experimental import pallas as pl
from jax.experimental.pallas import tpu as pltpu
from jax.experimental.pallas import tpu_sc as plsc

_BATCH = 16384
_NF = 26
_NC = 2
_NS = 16
_NW = _NC * _NS
_RPW = _BATCH // _NW
_IPW = _RPW * _NF
_CHUNKS = _RPW // 16


_NF_LO = 13


def _sc_body(xt_hbm, table_hbm, bias_hbm, out_hbm, idx_v, val_v, out_v, bias_v,
             sem, gsem, gsem2, bsem):
    wid = lax.axis_index("s") * _NC + lax.axis_index("c")
    base = wid * _RPW

    pltpu.async_copy(bias_hbm, bias_v.at[pl.ds(0, 1)], bsem)

    stage = [
        pltpu.async_copy(
            xt_hbm.at[f, pl.ds(base, _RPW)],
            idx_v.at[pl.ds(f * _RPW, _RPW)],
            sem,
        )
        for f in range(_NF)
    ]
    tbl = table_hbm.at[0]
    n_lo = _NF_LO * _RPW

    for d in stage[:_NF_LO]:
        d.wait()
    g_lo = pltpu.async_copy(
        tbl.at[idx_v.at[pl.ds(0, n_lo)]], val_v.at[pl.ds(0, n_lo)], gsem
    )
    for d in stage[_NF_LO:]:
        d.wait()
    g_hi = pltpu.async_copy(
        tbl.at[idx_v.at[pl.ds(n_lo, _IPW - n_lo)]],
        val_v.at[pl.ds(n_lo, _IPW - n_lo)],
        gsem2,
    )
    g_lo.wait()

    def lo_body(c, carry):
        acc = val_v[pl.ds(c * 16, 16)]
        for f in range(1, _NF_LO):
            acc = acc + val_v[pl.ds(f * _RPW + c * 16, 16)]
        out_v[pl.ds(c * 16, 16)] = acc
        return carry

    lax.fori_loop(0, _CHUNKS, lo_body, 0)
    g_hi.wait()
    pltpu.make_async_copy(bias_hbm, bias_v.at[pl.ds(0, 1)], bsem).wait()
    b = bias_v[...][0]

    def hi_body(c, carry):
        acc = out_v[pl.ds(c * 16, 16)] + b
        for f in range(_NF_LO, _NF):
            acc = acc + val_v[pl.ds(f * _RPW + c * 16, 16)]
        out_v[pl.ds(c * 16, 16)] = acc
        return carry

    lax.fori_loop(0, _CHUNKS, hi_body, 0)
    pltpu.sync_copy(out_v, out_hbm.at[pl.ds(base, _RPW)])


_sc_call = pl.kernel(
    _sc_body,
    out_type=jax.ShapeDtypeStruct((_BATCH,), jnp.float32),
    mesh=plsc.VectorSubcoreMesh(core_axis_name="c", subcore_axis_name="s"),
    scratch_types=[
        pltpu.VMEM((_IPW,), jnp.int32),
        pltpu.VMEM((_IPW,), jnp.float32),
        pltpu.VMEM((_RPW,), jnp.float32),
        pltpu.VMEM((16,), jnp.float32),
        pltpu.SemaphoreType.DMA,
        pltpu.SemaphoreType.DMA,
        pltpu.SemaphoreType.DMA,
        pltpu.SemaphoreType.DMA,
    ],
    compiler_params=pltpu.CompilerParams(needs_layout_passes=False),
)


@jax.jit
def kernel(x, table, bias):
    out = _sc_call(x.T, table.T, bias)
    return out.reshape(-1, 1)
